# Initial kernel scaffold; baseline (speedup 1.0000x reference)
import jax
import jax.numpy as jnp
from jax.experimental import pallas as pl


def kernel(t, z, W1, b1, W2, b2):
    raise NotImplementedError("write your pallas kernel here")


# speedup vs baseline: 4.6640x; 4.6640x over previous
import jax
import jax.numpy as jnp
from jax import lax
from jax.experimental import pallas as pl
from jax.experimental.pallas import tpu as pltpu

_INTEGRAL = 1.0
_BB = 512


def _cnf_kernel(s_ref, z_ref, w1_ref, b1_ref, w2_ref, b2_ref, f_ref, dl_ref):
    t = s_ref[0]
    scale = s_ref[1]
    z = z_ref[...]
    w1 = w1_ref[...]
    w1z = w1[1:, :]
    w2 = w2_ref[...]

    pre = jnp.dot(z, w1z, preferred_element_type=jnp.float32)
    pre = pre + t * w1[0:1, :] + b1_ref[...]
    h = jnp.tanh(pre)
    mlp = jnp.dot(h, w2, preferred_element_type=jnp.float32) + b2_ref[...]

    factor = -_INTEGRAL * t
    f_ref[...] = factor * (z - scale * mlp)

    g = jnp.dot(w2, w1z, preferred_element_type=jnp.float32)
    hh = g.shape[0]
    rows = lax.broadcasted_iota(jnp.int32, (hh, hh), 0)
    cols = lax.broadcasted_iota(jnp.int32, (hh, hh), 1)
    c = jnp.sum(jnp.where(rows == cols, g, 0.0), axis=0, keepdims=True)
    tr = jnp.sum((1.0 - h * h) * c, axis=1, keepdims=True)
    dl_ref[...] = -factor * (jnp.float32(z.shape[1]) - scale * tr)


def kernel(t, z, W1, b1, W2, b2):
    B, D = z.shape
    H = W2.shape[0]
    t0 = t[0].astype(jnp.float32)
    scale = 1.0 / jnp.sqrt(1.0 - jnp.exp(-(_INTEGRAL * t0 * t0)))
    scalars = jnp.stack([t0, scale])

    grid = (B // _BB,)
    f, dl = pl.pallas_call(
        _cnf_kernel,
        grid=grid,
        in_specs=[
            pl.BlockSpec(memory_space=pltpu.SMEM),
            pl.BlockSpec((_BB, D), lambda i: (i, 0)),
            pl.BlockSpec((D + 1, H), lambda i: (0, 0)),
            pl.BlockSpec((1, H), lambda i: (0, 0)),
            pl.BlockSpec((H, D), lambda i: (0, 0)),
            pl.BlockSpec((1, D), lambda i: (0, 0)),
        ],
        out_specs=[
            pl.BlockSpec((_BB, D), lambda i: (i, 0)),
            pl.BlockSpec((_BB, 1), lambda i: (i, 0)),
        ],
        out_shape=[
            jax.ShapeDtypeStruct((B, D), jnp.float32),
            jax.ShapeDtypeStruct((B, 1), jnp.float32),
        ],
        compiler_params=pltpu.CompilerParams(
            dimension_semantics=("parallel",),
        ),
        name="cnf_trace_fused",
    )(scalars, z, W1, b1.reshape(1, H), W2, b2.reshape(1, D))
    return f, dl


# speedup vs baseline: 4.8318x; 1.0360x over previous
import jax
import jax.numpy as jnp
from jax import lax
from jax.experimental import pallas as pl
from jax.experimental.pallas import tpu as pltpu

_INTEGRAL = 1.0
_BB = 512


def _cnf_kernel(t_ref, z_ref, w1_ref, b1_ref, w2_ref, b2_ref, f_ref, dl_ref):
    t = t_ref[0]
    z = z_ref[...]
    w1 = w1_ref[...]
    w1z = w1[1:, :]
    w2 = w2_ref[...]

    pre = jnp.dot(z, w1z, preferred_element_type=jnp.float32)
    pre = pre + t * w1[0:1, :] + b1_ref[...]
    h = jnp.tanh(pre)
    mlp = jnp.dot(h, w2, preferred_element_type=jnp.float32) + b2_ref[...]

    a = -_INTEGRAL * t
    tm = jnp.full((1, 1), t, dtype=jnp.float32)
    b = a * lax.rsqrt(1.0 - jnp.exp(-(_INTEGRAL * tm * tm)))
    f_ref[...] = a * z - b * mlp

    g = jnp.dot(w2, w1z, preferred_element_type=jnp.float32)
    hh = g.shape[0]
    rows = lax.broadcasted_iota(jnp.int32, (hh, hh), 0)
    cols = lax.broadcasted_iota(jnp.int32, (hh, hh), 1)
    c = jnp.sum(jnp.where(rows == cols, g, 0.0), axis=0, keepdims=True)
    tr = jnp.sum((1.0 - h * h) * c, axis=1, keepdims=True)
    dl_ref[...] = b * tr - a * jnp.float32(z.shape[1])


def kernel(t, z, W1, b1, W2, b2):
    B, D = z.shape
    H = W2.shape[0]

    grid = (B // _BB,)
    f, dl = pl.pallas_call(
        _cnf_kernel,
        grid=grid,
        in_specs=[
            pl.BlockSpec(memory_space=pltpu.SMEM),
            pl.BlockSpec((_BB, D), lambda i: (i, 0)),
            pl.BlockSpec((D + 1, H), lambda i: (0, 0)),
            pl.BlockSpec((1, H), lambda i: (0, 0)),
            pl.BlockSpec((H, D), lambda i: (0, 0)),
            pl.BlockSpec((1, D), lambda i: (0, 0)),
        ],
        out_specs=[
            pl.BlockSpec((_BB, D), lambda i: (i, 0)),
            pl.BlockSpec((_BB, 1), lambda i: (i, 0)),
        ],
        out_shape=[
            jax.ShapeDtypeStruct((B, D), jnp.float32),
            jax.ShapeDtypeStruct((B, 1), jnp.float32),
        ],
        compiler_params=pltpu.CompilerParams(
            dimension_semantics=("parallel",),
        ),
        name="cnf_trace_fused",
    )(t, z, W1, b1.reshape(1, H), W2, b2.reshape(1, D))
    return f, dl


# speedup vs baseline: 5.8087x; 1.2022x over previous
import jax
import jax.numpy as jnp
from jax import lax
from jax.experimental import pallas as pl
from jax.experimental.pallas import tpu as pltpu

_INTEGRAL = 1.0
_BB = 4096


def _cnf_kernel(t_ref, z_ref, w1_ref, b1_ref, w2_ref, b2_ref, f_ref, dl_ref):
    t = t_ref[0]
    z = z_ref[...]
    w1 = w1_ref[...]
    w1z = w1[1:, :]
    w2 = w2_ref[...]

    pre = jnp.dot(z, w1z, preferred_element_type=jnp.float32)
    pre = pre + t * w1[0:1, :] + b1_ref[...]
    h = jnp.tanh(pre)
    mlp = jnp.dot(h, w2, preferred_element_type=jnp.float32) + b2_ref[...]

    a = -_INTEGRAL * t
    tm = jnp.full((1, 1), t, dtype=jnp.float32)
    b = a * lax.rsqrt(1.0 - jnp.exp(-(_INTEGRAL * tm * tm)))
    f_ref[...] = a * z - b * mlp

    g = jnp.dot(w2, w1z, preferred_element_type=jnp.float32)
    hh = g.shape[0]
    rows = lax.broadcasted_iota(jnp.int32, (hh, hh), 0)
    cols = lax.broadcasted_iota(jnp.int32, (hh, hh), 1)
    c = jnp.sum(jnp.where(rows == cols, g, 0.0), axis=0, keepdims=True)
    tr = jnp.sum((1.0 - h * h) * c, axis=1, keepdims=True)
    dl_ref[...] = b * tr - a * jnp.float32(z.shape[1])


def kernel(t, z, W1, b1, W2, b2):
    B, D = z.shape
    H = W2.shape[0]

    grid = (B // _BB,)
    f, dl = pl.pallas_call(
        _cnf_kernel,
        grid=grid,
        in_specs=[
            pl.BlockSpec(memory_space=pltpu.SMEM),
            pl.BlockSpec((_BB, D), lambda i: (i, 0)),
            pl.BlockSpec((D + 1, H), lambda i: (0, 0)),
            pl.BlockSpec((1, H), lambda i: (0, 0)),
            pl.BlockSpec((H, D), lambda i: (0, 0)),
            pl.BlockSpec((1, D), lambda i: (0, 0)),
        ],
        out_specs=[
            pl.BlockSpec((_BB, D), lambda i: (i, 0)),
            pl.BlockSpec((_BB, 1), lambda i: (i, 0)),
        ],
        out_shape=[
            jax.ShapeDtypeStruct((B, D), jnp.float32),
            jax.ShapeDtypeStruct((B, 1), jnp.float32),
        ],
        compiler_params=pltpu.CompilerParams(
            dimension_semantics=("parallel",),
        ),
        name="cnf_trace_fused",
    )(t, z, W1, b1.reshape(1, H), W2, b2.reshape(1, D))
    return f, dl


# speedup vs baseline: 5.8266x; 1.0031x over previous
import jax
import jax.numpy as jnp
from jax import lax
from jax.experimental import pallas as pl
from jax.experimental.pallas import tpu as pltpu

_INTEGRAL = 1.0
_BB = 4096


def _cnf_kernel(t_ref, z_ref, w1_ref, b1_ref, w2_ref, b2_ref, f_ref, dl_ref):
    t = t_ref[0]
    z = z_ref[...]
    w1 = w1_ref[...]
    w1z = w1[1:, :]
    w2 = w2_ref[...]

    pre = jnp.dot(z, w1z, preferred_element_type=jnp.float32)
    pre = pre + t * w1[0:1, :] + b1_ref[...]
    h = jnp.tanh(pre)
    mlp = jnp.dot(h, w2, preferred_element_type=jnp.float32) + b2_ref[...]

    a = -_INTEGRAL * t
    tm = jnp.full((1, 1), t, dtype=jnp.float32)
    b = a * lax.rsqrt(1.0 - jnp.exp(-(_INTEGRAL * tm * tm)))
    f_ref[...] = a * z - b * mlp

    g = jnp.dot(w2, w1z, preferred_element_type=jnp.float32)
    hh = g.shape[0]
    rows = lax.broadcasted_iota(jnp.int32, (hh, hh), 0)
    cols = lax.broadcasted_iota(jnp.int32, (hh, hh), 1)
    c = jnp.sum(jnp.where(rows == cols, g, 0.0), axis=0, keepdims=True)
    c0 = jnp.sum(c, axis=1, keepdims=True)
    tr_neg = jnp.sum((h * h) * (b * c), axis=1, keepdims=True)
    dl_ref[...] = (b * c0 - a * jnp.float32(z.shape[1])) - tr_neg


def kernel(t, z, W1, b1, W2, b2):
    B, D = z.shape
    H = W2.shape[0]

    grid = (B // _BB,)
    f, dl = pl.pallas_call(
        _cnf_kernel,
        grid=grid,
        in_specs=[
            pl.BlockSpec(memory_space=pltpu.SMEM),
            pl.BlockSpec((_BB, D), lambda i: (i, 0)),
            pl.BlockSpec((D + 1, H), lambda i: (0, 0)),
            pl.BlockSpec((1, H), lambda i: (0, 0)),
            pl.BlockSpec((H, D), lambda i: (0, 0)),
            pl.BlockSpec((1, D), lambda i: (0, 0)),
        ],
        out_specs=[
            pl.BlockSpec((_BB, D), lambda i: (i, 0)),
            pl.BlockSpec((_BB, 1), lambda i: (i, 0)),
        ],
        out_shape=[
            jax.ShapeDtypeStruct((B, D), jnp.float32),
            jax.ShapeDtypeStruct((B, 1), jnp.float32),
        ],
        compiler_params=pltpu.CompilerParams(
            dimension_semantics=("parallel",),
        ),
        name="cnf_trace_fused",
    )(t, z, W1, b1.reshape(1, H), W2, b2.reshape(1, D))
    return f, dl


# speedup vs baseline: 5.9146x; 1.0151x over previous
import jax
import jax.numpy as jnp
from jax import lax
from jax.experimental import pallas as pl
from jax.experimental.pallas import tpu as pltpu

_INTEGRAL = 1.0
_BB = 2048


def _cnf_kernel(t_ref, z_ref, w1_ref, b1_ref, w2_ref, b2_ref, f_ref, dl_ref):
    t = t_ref[0]
    z = z_ref[...]
    w1 = w1_ref[...]
    w1z = w1[1:, :]
    w2 = w2_ref[...]

    pre = jnp.dot(z, w1z, preferred_element_type=jnp.float32)
    pre = pre + t * w1[0:1, :] + b1_ref[...]
    h = jnp.tanh(pre)
    mlp = jnp.dot(h, w2, preferred_element_type=jnp.float32) + b2_ref[...]

    a = -_INTEGRAL * t
    tm = jnp.full((1, 1), t, dtype=jnp.float32)
    b = a * lax.rsqrt(1.0 - jnp.exp(-(_INTEGRAL * tm * tm)))
    f_ref[...] = a * z - b * mlp

    g = jnp.dot(w2, w1z, preferred_element_type=jnp.float32)
    hh = g.shape[0]
    rows = lax.broadcasted_iota(jnp.int32, (hh, hh), 0)
    cols = lax.broadcasted_iota(jnp.int32, (hh, hh), 1)
    c = jnp.sum(jnp.where(rows == cols, g, 0.0), axis=0, keepdims=True)
    c0 = jnp.sum(c, axis=1, keepdims=True)
    tr_neg = jnp.sum((h * h) * (b * c), axis=1, keepdims=True)
    dl_ref[...] = (b * c0 - a * jnp.float32(z.shape[1])) - tr_neg


def kernel(t, z, W1, b1, W2, b2):
    B, D = z.shape
    H = W2.shape[0]

    grid = (B // _BB,)
    f, dl = pl.pallas_call(
        _cnf_kernel,
        grid=grid,
        in_specs=[
            pl.BlockSpec(memory_space=pltpu.SMEM),
            pl.BlockSpec((_BB, D), lambda i: (i, 0)),
            pl.BlockSpec((D + 1, H), lambda i: (0, 0)),
            pl.BlockSpec((1, H), lambda i: (0, 0)),
            pl.BlockSpec((H, D), lambda i: (0, 0)),
            pl.BlockSpec((1, D), lambda i: (0, 0)),
        ],
        out_specs=[
            pl.BlockSpec((_BB, D), lambda i: (i, 0)),
            pl.BlockSpec((_BB, 1), lambda i: (i, 0)),
        ],
        out_shape=[
            jax.ShapeDtypeStruct((B, D), jnp.float32),
            jax.ShapeDtypeStruct((B, 1), jnp.float32),
        ],
        compiler_params=pltpu.CompilerParams(
            dimension_semantics=("parallel",),
        ),
        name="cnf_trace_fused",
    )(t, z, W1, b1.reshape(1, H), W2, b2.reshape(1, D))
    return f, dl
